# baseline (speedup 1.0000x reference)
import functools

import jax
import jax.numpy as jnp
from jax import lax
from jax.experimental import pallas as pl
from jax.experimental.pallas import tpu as pltpu
from jax.experimental.pallas import tpu_sc as plsc

N_NODES = 10000
NPAD = 10240
NC, NS = 2, 16
SEG = NPAD // NS
K = 64
R = 1000
GRID = N_NODES // R

def _mesh():
    return plsc.VectorSubcoreMesh(core_axis_name="c", subcore_axis_name="s",
                                  num_cores=NC, num_subcores=NS)


def _pad_edges(e):
    quantum = NC * NS * K * CB * 2
    epad = ((e + quantum - 1) // quantum) * quantum
    return epad


def _deg_body(epad, dst_hbm, w_hbm, out_hbm, didx0, didx1, wbk0, wbk1,
              zb, acc, sem_st0, sem_st1, sem_s):
    c = lax.axis_index("c")
    s = lax.axis_index("s")
    nblk = epad // (CB * K * NC * NS)
    blk0 = (c * NS + s) * nblk

    def zero(i, _):
        zb[pl.ds(i * 16, 16)] = jnp.zeros((16,), jnp.float32)
        return 0

    lax.fori_loop(0, SEG // 16, zero, 0)
    pltpu.sync_copy(zb, acc.at[pl.ds(s * SEG, SEG)])
    plsc.subcore_barrier()

    slots = ((didx0, wbk0, sem_st0), (didx1, wbk1, sem_st1))

    def stage_issue(b, sl):
        row = (blk0 + b) * CB
        pltpu.async_copy(dst_hbm.at[pl.ds(row, CB)], sl[0], sl[2])
        pltpu.async_copy(w_hbm.at[pl.ds(row, CB)], sl[1], sl[2])

    def stage_wait(sl):
        pltpu.make_async_copy(dst_hbm.at[pl.ds(0, CB)], sl[0], sl[2]).wait()
        pltpu.make_async_copy(w_hbm.at[pl.ds(0, CB)], sl[1], sl[2]).wait()

    def do_block(sl):
        didx, wbk, _ = sl
        stage_wait(sl)
        ds_ = []
        for j in range(CB):
            ds_.append(pltpu.async_copy(wbk.at[j], acc.at[didx.at[j]],
                                        sem_s, add=True))
        for d in ds_:
            d.wait()

    stage_issue(0, slots[0])
    stage_issue(1, slots[1])

    def pair(i, _):
        do_block(slots[0])
        stage_issue(lax.rem(2 * i + 2, nblk), slots[0])
        do_block(slots[1])
        stage_issue(lax.rem(2 * i + 3, nblk), slots[1])
        return 0

    lax.fori_loop(0, nblk // 2, pair, 0)
    stage_wait(slots[0])
    stage_wait(slots[1])
    plsc.subcore_barrier()
    pltpu.sync_copy(acc.at[pl.ds(s * SEG, SEG)],
                    out_hbm.at[c, pl.ds(s * SEG, SEG)])


def _make_deg(epad):
    return pl.kernel(
        functools.partial(_deg_body, epad),
        out_type=jax.ShapeDtypeStruct((NC, NPAD), jnp.float32),
        mesh=_mesh(),
        scratch_types=[
            pltpu.VMEM((CB, K), jnp.int32),
            pltpu.VMEM((CB, K), jnp.int32),
            pltpu.VMEM((CB, K), jnp.float32),
            pltpu.VMEM((CB, K), jnp.float32),
            pltpu.VMEM((SEG,), jnp.float32),
            pltpu.VMEM_SHARED((NPAD,), jnp.float32),
            pltpu.SemaphoreType.DMA,
            pltpu.SemaphoreType.DMA,
            pltpu.SemaphoreType.DMA,
        ],
    )


CB = 8


def _prop_body(epad, f2, esplit, g_hbm, src_hbm, dst_hbm, w_hbm, out_hbm,
               sidx0, sidx1, didx0, didx1, wbk0, wbk1,
               rows0, rows1, rows2, rows3,
               dprime, dact, acc, sem_st0, sem_st1,
               sem_g0, sem_g1, sem_g2, sem_g3,
               sem_s0, sem_s1, sem_s2, sem_s3):
    c = lax.axis_index("c")
    s = lax.axis_index("s")
    nblk_tot = epad // (CB * K)
    if esplit:
        nblk = nblk_tot // (NC * NS)
        blk0 = (c * NS + s) * nblk
        cbase = None
    else:
        nblk = nblk_tot // NS
        blk0 = s * nblk
        cbase = c * N_NODES

    def zzero(i, _):
        for q in range(f2 // 16):
            z16 = jnp.zeros((16,), jnp.float32)
            rows0[i, pl.ds(q * 16, 16)] = z16
            rows1[i, pl.ds(q * 16, 16)] = z16
            rows2[i, pl.ds(q * 16, 16)] = z16
            rows3[i, pl.ds(q * 16, 16)] = z16
        return 0

    lax.fori_loop(0, K, zzero, 0)

    def zcopy(j, _):
        pltpu.sync_copy(rows0, acc.at[pl.ds(s * SEG + j * K, K)])
        return 0

    lax.fori_loop(0, SEG // K, zcopy, 0)
    plsc.subcore_barrier()

    rows = (rows0, rows1, rows2, rows3)
    semg = (sem_g0, sem_g1, sem_g2, sem_g3)
    sems = (sem_s0, sem_s1, sem_s2, sem_s3)
    slots = ((sidx0, didx0, wbk0, sem_st0), (sidx1, didx1, wbk1, sem_st1))

    iota16 = lax.iota(jnp.int32, 16)
    for q in range(4):
        for k in range(K // 16):
            dprime[q, pl.ds(k * 16, 16)] = (
                s * SEG + q * K + k * 16 + iota16)
    for q in range(4):
        pltpu.async_copy(rows[q], acc.at[dprime.at[q]], sems[q], add=True)

    def scat_wait(sem):
        pltpu.make_async_copy(rows0, acc.at[dprime.at[0]], sem).wait()

    def gath_wait(rb, sg):
        pltpu.make_async_copy(g_hbm.at[sidx0.at[0]], rb, sg).wait()

    def stage_issue(b, sl):
        row = (blk0 + b) * CB
        pltpu.async_copy(src_hbm.at[pl.ds(row, CB)], sl[0], sl[3])
        pltpu.async_copy(dst_hbm.at[pl.ds(row, CB)], sl[1], sl[3])
        pltpu.async_copy(w_hbm.at[pl.ds(row, CB)], sl[2], sl[3])

    def stage_wait_add(sl):
        pltpu.make_async_copy(src_hbm.at[pl.ds(0, CB)], sl[0], sl[3]).wait()
        pltpu.make_async_copy(dst_hbm.at[pl.ds(0, CB)], sl[1], sl[3]).wait()
        pltpu.make_async_copy(w_hbm.at[pl.ds(0, CB)], sl[2], sl[3]).wait()
        if cbase is not None:
            for j in range(CB):
                for k in range(K // 16):
                    sl[0][j, pl.ds(k * 16, 16)] = (
                        sl[0][j, pl.ds(k * 16, 16)] + cbase)

    def do_block(sl, nsl):
        sidx, didx, wbk, _ = sl
        for j in range(CB):
            rb, sg, ss = rows[j % 4], semg[j % 4], sems[j % 4]
            gath_wait(rb, sg)
            if j == CB - 3:
                stage_wait_add(nsl)
            tq = (j + 3) % 4
            scat_wait(sems[tq])
            ib = sidx.at[j + 3] if j < CB - 3 else nsl[0].at[j - (CB - 3)]
            pltpu.async_copy(g_hbm.at[ib], rows[tq], semg[tq])

            def scale(g, _):
                wg = wbk[j, pl.ds(g * 16, 16)]
                for jj in range(16):
                    wb = jnp.broadcast_to(
                        lax.slice_in_dim(wg, jj, jj + 1), (16,))
                    r = g * 16 + jj
                    for q in range(f2 // 16):
                        rb[r, pl.ds(q * 16, 16)] = (
                            rb[r, pl.ds(q * 16, 16)] * wb)
                return 0

            lax.fori_loop(0, K // 16, scale, 0)
            for k in range(K // 16):
                dact[j % 4, pl.ds(k * 16, 16)] = didx[j, pl.ds(k * 16, 16)]
            pltpu.async_copy(rb, acc.at[dact.at[j % 4]], ss, add=True)

    stage_issue(0, slots[0])
    stage_issue(1, slots[1])
    stage_wait_add(slots[0])
    for t in range(3):
        scat_wait(sems[t])
        pltpu.async_copy(g_hbm.at[sidx0.at[t]], rows[t], semg[t])

    def pair(i, _):
        do_block(slots[0], slots[1])
        stage_issue(lax.rem(2 * i + 2, nblk), slots[0])
        do_block(slots[1], slots[0])
        stage_issue(lax.rem(2 * i + 3, nblk), slots[1])
        return 0

    lax.fori_loop(0, nblk // 2, pair, 0)
    stage_wait_add(slots[1])
    for t in range(3):
        gath_wait(rows[t], semg[t])
    scat_wait(sems[3])
    plsc.subcore_barrier()

    pltpu.sync_copy(acc.at[pl.ds(s * SEG, SEG)],
                    out_hbm.at[c, pl.ds(s * SEG, SEG)])


def _make_prop(epad, f2, esplit):
    return pl.kernel(
        functools.partial(_prop_body, epad, f2, esplit),
        out_type=jax.ShapeDtypeStruct((NC, NPAD, f2), jnp.float32),
        mesh=_mesh(),
        scratch_types=[
            pltpu.VMEM((CB, K), jnp.int32),
            pltpu.VMEM((CB, K), jnp.int32),
            pltpu.VMEM((CB, K), jnp.int32),
            pltpu.VMEM((CB, K), jnp.int32),
            pltpu.VMEM((CB, K), jnp.float32),
            pltpu.VMEM((CB, K), jnp.float32),
            pltpu.VMEM((K, f2), jnp.float32),
            pltpu.VMEM((K, f2), jnp.float32),
            pltpu.VMEM((K, f2), jnp.float32),
            pltpu.VMEM((K, f2), jnp.float32),
            pltpu.VMEM((4, K), jnp.int32),
            pltpu.VMEM((4, K), jnp.int32),
            pltpu.VMEM_SHARED((NPAD, f2), jnp.float32),
        ] + [pltpu.SemaphoreType.DMA] * 10,
    )


def _tc_dinv(deg_ref, dinv_ref):
    d = jnp.sum(deg_ref[...], axis=0) + 1.0
    dv = lax.rsqrt(d)
    dinv_ref[...] = jnp.broadcast_to(dv[:, None], (1024, 128))


def _tc1(x_ref, w1_ref, dinv_ref, o_ref):
    h = jnp.dot(x_ref[...], w1_ref[...], preferred_element_type=jnp.float32)
    g = h * dinv_ref[:, :1]
    o_ref[0] = g[:, :128]
    o_ref[1] = g[:, 128:]


def _tc_mid(s_ref, g_ref, dinv_ref, b_ref, w_ref, o_ref, *, split):
    dv = dinv_ref[:, :1]
    p = (jnp.concatenate([s_ref[0], s_ref[1]], axis=1)
         + jnp.concatenate([g_ref[0], g_ref[1]], axis=1))
    h = jnp.maximum(dv * p + b_ref[...], 0.0)
    g = jnp.dot(h, w_ref[...], preferred_element_type=jnp.float32) * dv
    if split:
        o_ref[0] = g[:, :128]
        o_ref[1] = g[:, 128:]
    else:
        o_ref[...] = g


def _tc4(s_ref, g_ref, dinv_ref, b_ref, h3_ref, o_ref):
    dv = dinv_ref[:, :1]
    p = s_ref[0] + s_ref[1] + g_ref[...]
    h3 = dv * p + b_ref[...]
    h3_ref[...] = h3
    o_ref[...] = dv * h3


def _tc5(s_ref, g_ref, h3_ref, dinv_ref, o_ref):
    dv = dinv_ref[:, :1]
    h4 = dv * (s_ref[0] + s_ref[1] + g_ref[...])
    out = 0.8 * h4 + 0.2 * h3_ref[...]
    o_ref[...] = jnp.where(out >= 0.0, out, 0.01 * out)


def _row_spec(width):
    return pl.BlockSpec((R, width), lambda i: (i, 0))


def _half_spec(width):
    return pl.BlockSpec((2, R, width), lambda i: (0, i, 0))


def _full_spec(shape):
    nd = len(shape)
    return pl.BlockSpec(shape, lambda i, _n=nd: (0,) * _n)


def kernel(x, edge_index, edge_weight, W1, b1, W2, b2, W3, b3):
    e = edge_weight.shape[0]
    epad = _pad_edges(e)
    pad = epad - e
    spread = jnp.arange(pad, dtype=jnp.int32)
    src = jnp.concatenate([edge_index[0], spread % N_NODES])
    dst = jnp.concatenate([edge_index[1],
                           N_NODES + (spread % (NPAD - N_NODES))])
    w = jnp.concatenate([edge_weight, jnp.zeros((pad,), jnp.float32)])
    src2 = src.reshape(epad // K, K)
    dst2 = dst.reshape(epad // K, K)
    w2 = w.reshape(epad // K, K)
    b1r, b2r, b3r = (b.reshape(1, -1) for b in (b1, b2, b3))

    deg_p = _make_deg(epad)(dst2, w2)

    dinv = pl.pallas_call(
        _tc_dinv,
        grid=(NPAD // 1024,),
        in_specs=[pl.BlockSpec((NC, 1024), lambda i: (0, i))],
        out_specs=pl.BlockSpec((1024, 128), lambda i: (i, 0)),
        out_shape=jax.ShapeDtypeStruct((NPAD, 128), jnp.float32),
    )(deg_p)

    prop_fs = _make_prop(epad, 128, False)
    prop_es = _make_prop(epad, 128, True)

    g1 = pl.pallas_call(
        _tc1,
        grid=(GRID,),
        in_specs=[_row_spec(128), _full_spec((128, 256)), _row_spec(128)],
        out_specs=_half_spec(128),
        out_shape=jax.ShapeDtypeStruct((2, N_NODES, 128), jnp.float32),
    )(x, W1, dinv)

    s1 = prop_fs(g1.reshape(2 * N_NODES, 128), src2, dst2, w2)

    g2 = pl.pallas_call(
        functools.partial(_tc_mid, split=True),
        grid=(GRID,),
        in_specs=[_half_spec(128), _half_spec(128), _row_spec(128),
                  _full_spec((1, 256)), _full_spec((256, 256))],
        out_specs=_half_spec(128),
        out_shape=jax.ShapeDtypeStruct((2, N_NODES, 128), jnp.float32),
    )(s1, g1, dinv, b1r, W2)

    s2 = prop_fs(g2.reshape(2 * N_NODES, 128), src2, dst2, w2)

    g3 = pl.pallas_call(
        functools.partial(_tc_mid, split=False),
        grid=(GRID,),
        in_specs=[_half_spec(128), _half_spec(128), _row_spec(128),
                  _full_spec((1, 256)), _full_spec((256, 128))],
        out_specs=_row_spec(128),
        out_shape=jax.ShapeDtypeStruct((N_NODES, 128), jnp.float32),
    )(s2, g2, dinv, b2r, W3)

    s3 = prop_es(g3, src2, dst2, w2)

    h3, g4 = pl.pallas_call(
        _tc4,
        grid=(GRID,),
        in_specs=[_half_spec(128), _row_spec(128), _row_spec(128),
                  _full_spec((1, 128))],
        out_specs=[_row_spec(128), _row_spec(128)],
        out_shape=[jax.ShapeDtypeStruct((N_NODES, 128), jnp.float32),
                   jax.ShapeDtypeStruct((N_NODES, 128), jnp.float32)],
    )(s3, g3, dinv, b3r)

    s4 = prop_es(g4, src2, dst2, w2)

    out = pl.pallas_call(
        _tc5,
        grid=(GRID,),
        in_specs=[_half_spec(128), _row_spec(128), _row_spec(128),
                  _row_spec(128)],
        out_specs=_row_spec(128),
        out_shape=jax.ShapeDtypeStruct((N_NODES, 128), jnp.float32),
    )(s4, g4, h3, dinv)

    return out


# speedup vs baseline: 1.0050x; 1.0050x over previous
import functools

import jax
import jax.numpy as jnp
from jax import lax
from jax.experimental import pallas as pl
from jax.experimental.pallas import tpu as pltpu
from jax.experimental.pallas import tpu_sc as plsc

N_NODES = 10000
NPAD = 10240
NC, NS = 2, 16
SEG = NPAD // NS
K = 64
R = 1000
GRID = N_NODES // R

def _mesh():
    return plsc.VectorSubcoreMesh(core_axis_name="c", subcore_axis_name="s",
                                  num_cores=NC, num_subcores=NS)


def _pad_edges(e):
    quantum = NC * NS * K * CB * 2
    epad = ((e + quantum - 1) // quantum) * quantum
    return epad


def _deg_body(epad, dst_hbm, w_hbm, out_hbm, didx0, didx1, wbk0, wbk1,
              zb, acc, sem_st0, sem_st1, sem_s):
    c = lax.axis_index("c")
    s = lax.axis_index("s")
    nblk = epad // (CB * K * NC * NS)
    blk0 = (c * NS + s) * nblk

    def zero(i, _):
        zb[pl.ds(i * 16, 16)] = jnp.zeros((16,), jnp.float32)
        return 0

    lax.fori_loop(0, SEG // 16, zero, 0)
    pltpu.sync_copy(zb, acc.at[pl.ds(s * SEG, SEG)])
    plsc.subcore_barrier()

    slots = ((didx0, wbk0, sem_st0), (didx1, wbk1, sem_st1))

    def stage_issue(b, sl):
        row = (blk0 + b) * CB
        pltpu.async_copy(dst_hbm.at[pl.ds(row, CB)], sl[0], sl[2])
        pltpu.async_copy(w_hbm.at[pl.ds(row, CB)], sl[1], sl[2])

    def stage_wait(sl):
        pltpu.make_async_copy(dst_hbm.at[pl.ds(0, CB)], sl[0], sl[2]).wait()
        pltpu.make_async_copy(w_hbm.at[pl.ds(0, CB)], sl[1], sl[2]).wait()

    def do_block(sl):
        didx, wbk, _ = sl
        stage_wait(sl)
        ds_ = []
        for j in range(CB):
            ds_.append(pltpu.async_copy(wbk.at[j], acc.at[didx.at[j]],
                                        sem_s, add=True))
        for d in ds_:
            d.wait()

    stage_issue(0, slots[0])
    stage_issue(1, slots[1])

    def pair(i, _):
        do_block(slots[0])
        stage_issue(lax.rem(2 * i + 2, nblk), slots[0])
        do_block(slots[1])
        stage_issue(lax.rem(2 * i + 3, nblk), slots[1])
        return 0

    lax.fori_loop(0, nblk // 2, pair, 0)
    stage_wait(slots[0])
    stage_wait(slots[1])
    plsc.subcore_barrier()
    pltpu.sync_copy(acc.at[pl.ds(s * SEG, SEG)],
                    out_hbm.at[c, pl.ds(s * SEG, SEG)])


def _make_deg(epad):
    return pl.kernel(
        functools.partial(_deg_body, epad),
        out_type=jax.ShapeDtypeStruct((NC, NPAD), jnp.float32),
        mesh=_mesh(),
        scratch_types=[
            pltpu.VMEM((CB, K), jnp.int32),
            pltpu.VMEM((CB, K), jnp.int32),
            pltpu.VMEM((CB, K), jnp.float32),
            pltpu.VMEM((CB, K), jnp.float32),
            pltpu.VMEM((SEG,), jnp.float32),
            pltpu.VMEM_SHARED((NPAD,), jnp.float32),
            pltpu.SemaphoreType.DMA,
            pltpu.SemaphoreType.DMA,
            pltpu.SemaphoreType.DMA,
        ],
    )


CB = 8


def _prop_body(epad, f2, esplit, g_hbm, src_hbm, dst_hbm, w_hbm, out_hbm,
               sidx0, sidx1, didx0, didx1, wbk0, wbk1,
               rows0, rows1, rows2, rows3,
               dprime, dact, acc, sem_st0, sem_st1,
               sem_g0, sem_g1, sem_g2, sem_g3,
               sem_s0, sem_s1, sem_s2, sem_s3):
    c = lax.axis_index("c")
    s = lax.axis_index("s")
    nblk_tot = epad // (CB * K)
    if esplit:
        nblk = nblk_tot // (NC * NS)
        blk0 = (c * NS + s) * nblk
        cbase = None
    else:
        nblk = nblk_tot // NS
        blk0 = s * nblk
        cbase = c * N_NODES

    def zzero(i, _):
        for q in range(f2 // 16):
            rows0[i, pl.ds(q * 16, 16)] = jnp.zeros((16,), jnp.float32)
        return 0

    lax.fori_loop(0, K, zzero, 0)

    def zcopy(j, _):
        pltpu.sync_copy(rows0, acc.at[pl.ds(s * SEG + j * K, K)])
        return 0

    lax.fori_loop(0, SEG // K, zcopy, 0)
    plsc.subcore_barrier()

    rows = (rows0, rows1, rows2, rows3)
    semg = (sem_g0, sem_g1, sem_g2, sem_g3)
    sems = (sem_s0, sem_s1, sem_s2, sem_s3)
    slots = ((sidx0, didx0, wbk0, sem_st0), (sidx1, didx1, wbk1, sem_st1))

    iota16 = lax.iota(jnp.int32, 16)
    for k in range(K // 16):
        dprime[0, pl.ds(k * 16, 16)] = N_NODES + k * 16 + iota16
    for q in range(4):
        pltpu.async_copy(rows[q], acc.at[dprime.at[0]], sems[q], add=True)

    def scat_wait(sem):
        pltpu.make_async_copy(rows0, acc.at[dprime.at[0]], sem).wait()

    def gath_wait(rb, sg):
        pltpu.make_async_copy(g_hbm.at[sidx0.at[0]], rb, sg).wait()

    def stage_issue(b, sl):
        row = (blk0 + b) * CB
        pltpu.async_copy(src_hbm.at[pl.ds(row, CB)], sl[0], sl[3])
        pltpu.async_copy(dst_hbm.at[pl.ds(row, CB)], sl[1], sl[3])
        pltpu.async_copy(w_hbm.at[pl.ds(row, CB)], sl[2], sl[3])

    def stage_wait_add(sl):
        pltpu.make_async_copy(src_hbm.at[pl.ds(0, CB)], sl[0], sl[3]).wait()
        pltpu.make_async_copy(dst_hbm.at[pl.ds(0, CB)], sl[1], sl[3]).wait()
        pltpu.make_async_copy(w_hbm.at[pl.ds(0, CB)], sl[2], sl[3]).wait()
        if cbase is not None:
            for j in range(CB):
                for k in range(K // 16):
                    sl[0][j, pl.ds(k * 16, 16)] = (
                        sl[0][j, pl.ds(k * 16, 16)] + cbase)

    def do_block(sl, nsl):
        sidx, didx, wbk, _ = sl
        for j in range(CB):
            rb, sg, ss = rows[j % 4], semg[j % 4], sems[j % 4]
            gath_wait(rb, sg)
            if j == CB - 3:
                stage_wait_add(nsl)
            tq = (j + 3) % 4
            scat_wait(sems[tq])
            ib = sidx.at[j + 3] if j < CB - 3 else nsl[0].at[j - (CB - 3)]
            pltpu.async_copy(g_hbm.at[ib], rows[tq], semg[tq])

            def scale(g, _):
                wg = wbk[j, pl.ds(g * 16, 16)]
                for jj in range(16):
                    wb = jnp.broadcast_to(
                        lax.slice_in_dim(wg, jj, jj + 1), (16,))
                    r = g * 16 + jj
                    for q in range(f2 // 16):
                        rb[r, pl.ds(q * 16, 16)] = (
                            rb[r, pl.ds(q * 16, 16)] * wb)
                return 0

            lax.fori_loop(0, K // 16, scale, 0)
            for k in range(K // 16):
                dact[j % 4, pl.ds(k * 16, 16)] = didx[j, pl.ds(k * 16, 16)]
            pltpu.async_copy(rb, acc.at[dact.at[j % 4]], ss, add=True)

    stage_issue(0, slots[0])
    stage_issue(1, slots[1])
    stage_wait_add(slots[0])
    for t in range(3):
        scat_wait(sems[t])
        pltpu.async_copy(g_hbm.at[sidx0.at[t]], rows[t], semg[t])

    def pair(i, _):
        do_block(slots[0], slots[1])
        stage_issue(lax.rem(2 * i + 2, nblk), slots[0])
        do_block(slots[1], slots[0])
        stage_issue(lax.rem(2 * i + 3, nblk), slots[1])
        return 0

    lax.fori_loop(0, nblk // 2, pair, 0)
    stage_wait_add(slots[1])
    for t in range(3):
        gath_wait(rows[t], semg[t])
    scat_wait(sems[3])
    plsc.subcore_barrier()

    pltpu.sync_copy(acc.at[pl.ds(s * SEG, SEG)],
                    out_hbm.at[c, pl.ds(s * SEG, SEG)])


def _make_prop(epad, f2, esplit):
    return pl.kernel(
        functools.partial(_prop_body, epad, f2, esplit),
        out_type=jax.ShapeDtypeStruct((NC, NPAD, f2), jnp.float32),
        mesh=_mesh(),
        scratch_types=[
            pltpu.VMEM((CB, K), jnp.int32),
            pltpu.VMEM((CB, K), jnp.int32),
            pltpu.VMEM((CB, K), jnp.int32),
            pltpu.VMEM((CB, K), jnp.int32),
            pltpu.VMEM((CB, K), jnp.float32),
            pltpu.VMEM((CB, K), jnp.float32),
            pltpu.VMEM((K, f2), jnp.float32),
            pltpu.VMEM((K, f2), jnp.float32),
            pltpu.VMEM((K, f2), jnp.float32),
            pltpu.VMEM((K, f2), jnp.float32),
            pltpu.VMEM((1, K), jnp.int32),
            pltpu.VMEM((4, K), jnp.int32),
            pltpu.VMEM_SHARED((NPAD, f2), jnp.float32),
        ] + [pltpu.SemaphoreType.DMA] * 10,
    )


def _tc_dinv(deg_ref, dinv_ref):
    d = jnp.sum(deg_ref[...], axis=0) + 1.0
    dv = lax.rsqrt(d)
    dinv_ref[...] = jnp.broadcast_to(dv[:, None], (1024, 128))


def _tc1(x_ref, w1_ref, dinv_ref, o_ref):
    h = jnp.dot(x_ref[...], w1_ref[...], preferred_element_type=jnp.float32)
    g = h * dinv_ref[:, :1]
    o_ref[0] = g[:, :128]
    o_ref[1] = g[:, 128:]


def _tc_mid(s_ref, g_ref, dinv_ref, b_ref, w_ref, o_ref, *, split):
    dv = dinv_ref[:, :1]
    p = (jnp.concatenate([s_ref[0], s_ref[1]], axis=1)
         + jnp.concatenate([g_ref[0], g_ref[1]], axis=1))
    h = jnp.maximum(dv * p + b_ref[...], 0.0)
    g = jnp.dot(h, w_ref[...], preferred_element_type=jnp.float32) * dv
    if split:
        o_ref[0] = g[:, :128]
        o_ref[1] = g[:, 128:]
    else:
        o_ref[...] = g


def _tc4(s_ref, g_ref, dinv_ref, b_ref, h3_ref, o_ref):
    dv = dinv_ref[:, :1]
    p = s_ref[0] + s_ref[1] + g_ref[...]
    h3 = dv * p + b_ref[...]
    h3_ref[...] = h3
    o_ref[...] = dv * h3


def _tc5(s_ref, g_ref, h3_ref, dinv_ref, o_ref):
    dv = dinv_ref[:, :1]
    h4 = dv * (s_ref[0] + s_ref[1] + g_ref[...])
    out = 0.8 * h4 + 0.2 * h3_ref[...]
    o_ref[...] = jnp.where(out >= 0.0, out, 0.01 * out)


def _row_spec(width):
    return pl.BlockSpec((R, width), lambda i: (i, 0))


def _half_spec(width):
    return pl.BlockSpec((2, R, width), lambda i: (0, i, 0))


def _full_spec(shape):
    nd = len(shape)
    return pl.BlockSpec(shape, lambda i, _n=nd: (0,) * _n)


def kernel(x, edge_index, edge_weight, W1, b1, W2, b2, W3, b3):
    e = edge_weight.shape[0]
    epad = _pad_edges(e)
    pad = epad - e
    spread = jnp.arange(pad, dtype=jnp.int32)
    src = jnp.concatenate([edge_index[0], spread % N_NODES])
    dst = jnp.concatenate([edge_index[1],
                           N_NODES + (spread % (NPAD - N_NODES))])
    w = jnp.concatenate([edge_weight, jnp.zeros((pad,), jnp.float32)])
    src2 = src.reshape(epad // K, K)
    dst2 = dst.reshape(epad // K, K)
    w2 = w.reshape(epad // K, K)
    b1r, b2r, b3r = (b.reshape(1, -1) for b in (b1, b2, b3))

    deg_p = _make_deg(epad)(dst2, w2)

    dinv = pl.pallas_call(
        _tc_dinv,
        grid=(NPAD // 1024,),
        in_specs=[pl.BlockSpec((NC, 1024), lambda i: (0, i))],
        out_specs=pl.BlockSpec((1024, 128), lambda i: (i, 0)),
        out_shape=jax.ShapeDtypeStruct((NPAD, 128), jnp.float32),
    )(deg_p)

    prop_fs = _make_prop(epad, 128, False)
    prop_es = _make_prop(epad, 128, True)

    g1 = pl.pallas_call(
        _tc1,
        grid=(GRID,),
        in_specs=[_row_spec(128), _full_spec((128, 256)), _row_spec(128)],
        out_specs=_half_spec(128),
        out_shape=jax.ShapeDtypeStruct((2, N_NODES, 128), jnp.float32),
    )(x, W1, dinv)

    s1 = prop_fs(g1.reshape(2 * N_NODES, 128), src2, dst2, w2)

    g2 = pl.pallas_call(
        functools.partial(_tc_mid, split=True),
        grid=(GRID,),
        in_specs=[_half_spec(128), _half_spec(128), _row_spec(128),
                  _full_spec((1, 256)), _full_spec((256, 256))],
        out_specs=_half_spec(128),
        out_shape=jax.ShapeDtypeStruct((2, N_NODES, 128), jnp.float32),
    )(s1, g1, dinv, b1r, W2)

    s2 = prop_fs(g2.reshape(2 * N_NODES, 128), src2, dst2, w2)

    g3 = pl.pallas_call(
        functools.partial(_tc_mid, split=False),
        grid=(GRID,),
        in_specs=[_half_spec(128), _half_spec(128), _row_spec(128),
                  _full_spec((1, 256)), _full_spec((256, 128))],
        out_specs=_row_spec(128),
        out_shape=jax.ShapeDtypeStruct((N_NODES, 128), jnp.float32),
    )(s2, g2, dinv, b2r, W3)

    s3 = prop_es(g3, src2, dst2, w2)

    h3, g4 = pl.pallas_call(
        _tc4,
        grid=(GRID,),
        in_specs=[_half_spec(128), _row_spec(128), _row_spec(128),
                  _full_spec((1, 128))],
        out_specs=[_row_spec(128), _row_spec(128)],
        out_shape=[jax.ShapeDtypeStruct((N_NODES, 128), jnp.float32),
                   jax.ShapeDtypeStruct((N_NODES, 128), jnp.float32)],
    )(s3, g3, dinv, b3r)

    s4 = prop_es(g4, src2, dst2, w2)

    out = pl.pallas_call(
        _tc5,
        grid=(GRID,),
        in_specs=[_half_spec(128), _row_spec(128), _row_spec(128),
                  _row_spec(128)],
        out_specs=_row_spec(128),
        out_shape=jax.ShapeDtypeStruct((N_NODES, 128), jnp.float32),
    )(s4, g4, h3, dinv)

    return out
